# baseline (speedup 1.0000x reference)
import jax
import jax.numpy as jnp
from jax import lax
from jax.experimental import pallas as pl
from jax.experimental.pallas import tpu as pltpu
from jax.experimental.pallas import tpu_sc as plsc

N = 10000
E = 320000
D = 128
HID = 64
DOUT = 32
NG = 64

NC, NS = 2, 16
NW = NC * NS
HALF = 5120
EB = 64
ET = 344064
ER = ET // 128
CAP = ET // NW
CROWS = CAP // 128
NACC = 2 * HALF
STRIPE = HALF // NS
PADV = 2 * HALF - 1

_f32 = jnp.float32
_i32 = jnp.int32


def _selu(x):
    return 1.0507009873554805 * jnp.where(
        x > 0, x, 1.6732632423543772 * (jnp.exp(x) - 1.0))


def _interleave_bf16(h):
    t = h.reshape(h.shape[0], 4, 2, 16)
    return t.transpose(0, 1, 3, 2).reshape(h.shape[0], D).astype(
        jnp.bfloat16)


def _mm_body(x_ref, w_ref, a2_ref, h_ref, p_ref):
    i = pl.program_id(0)
    h = jnp.dot(x_ref[...], w_ref[...], preferred_element_type=_f32)
    h_ref[...] = _interleave_bf16(h)
    p = jnp.dot(h, a2_ref[...], preferred_element_type=_f32)
    row = i * x_ref.shape[0] + lax.broadcasted_iota(_i32, p.shape, 0)
    p_ref[...] = jnp.where(row < N, p, -1e30)


def _mm(x, W, A2):
    BR = 1024
    return pl.pallas_call(
        _mm_body,
        grid=(NACC // BR,),
        in_specs=[
            pl.BlockSpec((BR, D), lambda i: (i, 0)),
            pl.BlockSpec((D, D), lambda i: (0, 0)),
            pl.BlockSpec((D, 2), lambda i: (0, 0)),
        ],
        out_specs=[
            pl.BlockSpec((BR, D), lambda i: (i, 0)),
            pl.BlockSpec((BR, 2), lambda i: (i, 0)),
        ],
        out_shape=[
            jax.ShapeDtypeStruct((NACC, D), jnp.bfloat16),
            jax.ShapeDtypeStruct((NACC, 2), _f32),
        ],
    )(x, W, A2)


def _mid_body(acc_ref, d_ref, b_ref, w_ref, a2_ref, h_ref, p_ref):
    i = pl.program_id(0)
    den = d_ref[...] + 1e-16
    z = _selu(acc_ref[...] / den + b_ref[...])
    row = i * acc_ref.shape[0] + lax.broadcasted_iota(_i32, z.shape, 0)
    z = jnp.where(row < N, z, 0.0)
    h = jnp.dot(z, w_ref[...], preferred_element_type=_f32)
    h_ref[...] = _interleave_bf16(h)
    p = jnp.dot(h, a2_ref[...], preferred_element_type=_f32)
    prow = i * z.shape[0] + lax.broadcasted_iota(_i32, p.shape, 0)
    p_ref[...] = jnp.where(prow < N, p, -1e30)


def _mid(acc, den, b, W, A2):
    BR = 1024
    return pl.pallas_call(
        _mid_body,
        grid=(NACC // BR,),
        in_specs=[
            pl.BlockSpec((BR, D), lambda i: (i, 0)),
            pl.BlockSpec((BR, 1), lambda i: (i, 0)),
            pl.BlockSpec((1, D), lambda i: (0, 0)),
            pl.BlockSpec((D, D), lambda i: (0, 0)),
            pl.BlockSpec((D, 2), lambda i: (0, 0)),
        ],
        out_specs=[
            pl.BlockSpec((BR, D), lambda i: (i, 0)),
            pl.BlockSpec((BR, 2), lambda i: (i, 0)),
        ],
        out_shape=[
            jax.ShapeDtypeStruct((NACC, D), jnp.bfloat16),
            jax.ShapeDtypeStruct((NACC, 2), _f32),
        ],
    )(acc, den.reshape(NACC, 1), b.reshape(1, D), W, A2)


def _tail_body(acc_ref, d_ref, bb_ref, b_ref,
               wl1_ref, bl1_ref, wl2_ref, bl2_ref, out_ref,
               sum_ref, cnt_ref):
    i = pl.program_id(0)
    BR = acc_ref.shape[0]
    den = d_ref[...] + 1e-16
    z = _selu(acc_ref[...] / den + b_ref[...])
    row = i * BR + lax.broadcasted_iota(_i32, z.shape, 0)
    z = jnp.where(row < N, z, 0.0)

    bb = bb_ref[...].reshape(1, BR)
    gid = lax.broadcasted_iota(_i32, (NG, BR), 0)
    msk = (bb == gid).astype(_f32)

    @pl.when(i == 0)
    def _():
        sum_ref[...] = jnp.zeros_like(sum_ref)
        cnt_ref[...] = jnp.zeros_like(cnt_ref)

    sum_ref[...] += jnp.dot(msk, z, preferred_element_type=_f32)
    cnt_ref[...] += jnp.sum(msk, axis=1, keepdims=True)

    @pl.when(i == pl.num_programs(0) - 1)
    def _():
        g = sum_ref[...] / jnp.maximum(cnt_ref[...], 1.0)
        g = _selu(g)
        g = _selu(jnp.dot(g, wl1_ref[...], preferred_element_type=_f32)
                  + bl1_ref[...])
        lg = jnp.dot(g, wl2_ref[...], preferred_element_type=_f32) \
            + bl2_ref[...]
        m = jnp.max(lg, axis=1, keepdims=True)
        lse = jnp.log(jnp.sum(jnp.exp(lg - m), axis=1, keepdims=True))
        out_ref[...] = lg - m - lse


def _tail(acc, den, bb2d, b, Wl1, bl1, Wl2, bl2):
    BR = 1024
    return pl.pallas_call(
        _tail_body,
        grid=(NACC // BR,),
        in_specs=[
            pl.BlockSpec((BR, D), lambda i: (i, 0)),
            pl.BlockSpec((BR, 1), lambda i: (i, 0)),
            pl.BlockSpec((8, 128), lambda i: (i, 0)),
            pl.BlockSpec((1, D), lambda i: (0, 0)),
            pl.BlockSpec((D, HID), lambda i: (0, 0)),
            pl.BlockSpec((1, HID), lambda i: (0, 0)),
            pl.BlockSpec((HID, DOUT), lambda i: (0, 0)),
            pl.BlockSpec((1, DOUT), lambda i: (0, 0)),
        ],
        out_specs=pl.BlockSpec((NG, DOUT), lambda i: (0, 0)),
        out_shape=jax.ShapeDtypeStruct((NG, DOUT), _f32),
        scratch_shapes=[
            pltpu.VMEM((NG, D), _f32),
            pltpu.VMEM((NG, 1), _f32),
        ],
    )(acc, den.reshape(NACC, 1), bb2d, b.reshape(1, D),
      Wl1, bl1.reshape(1, HID), Wl2, bl2.reshape(1, DOUT))


def _part_body(s_hbm, d_hbm, sp_out, dp_out, cnt_out,
               s_in, d_in, ls0, ld0, ls1, ld1, cbuf):
    c = lax.axis_index("c")
    s = lax.axis_index("s")
    w = c * NS + s

    pltpu.sync_copy(s_hbm.at[pl.ds(w * CROWS, CROWS)], s_in)
    pltpu.sync_copy(d_hbm.at[pl.ds(w * CROWS, CROWS)], d_in)

    padsrc = jnp.full((16,), PADV, _i32)

    def init(k, _):
        spread = (k % (HALF // 16)) * 16 + jnp.arange(16, dtype=_i32)
        ls0[pl.ds(k * 16, 16)] = padsrc
        ls1[pl.ds(k * 16, 16)] = padsrc
        ld0[pl.ds(k * 16, 16)] = spread
        ld1[pl.ds(k * 16, 16)] = spread
        return 0
    lax.fori_loop(0, CAP // 16, init, 0)

    def compact(k, carry):
        off0, off1 = carry
        s16 = s_in[k // 8, pl.ds((k % 8) * 16, 16)]
        d16 = d_in[k // 8, pl.ds((k % 8) * 16, 16)]
        m0 = d16 < HALF
        m1 = jnp.logical_not(m0)
        plsc.store_compressed(ls0.at[pl.ds(off0, 16)], s16, mask=m0)
        plsc.store_compressed(ld0.at[pl.ds(off0, 16)], d16, mask=m0)
        plsc.store_compressed(ls1.at[pl.ds(off1, 16)], s16, mask=m1)
        plsc.store_compressed(ld1.at[pl.ds(off1, 16)], d16 - HALF, mask=m1)
        n0 = plsc.all_reduce_population_count(m0)[0]
        return (off0 + n0, off1 + (16 - n0))
    off0, off1 = lax.fori_loop(0, CAP // 16, compact, (0, 0))

    pltpu.sync_copy(ls0.at[pl.ds(0, CAP)], sp_out.at[pl.ds(w * CAP, CAP)])
    pltpu.sync_copy(ld0.at[pl.ds(0, CAP)], dp_out.at[pl.ds(w * CAP, CAP)])
    pltpu.sync_copy(ls1.at[pl.ds(0, CAP)],
                    sp_out.at[pl.ds((NW + w) * CAP, CAP)])
    pltpu.sync_copy(ld1.at[pl.ds(0, CAP)],
                    dp_out.at[pl.ds((NW + w) * CAP, CAP)])

    cbuf[pl.ds(0, 16)] = jnp.full((16,), off0, _i32)
    pltpu.sync_copy(cbuf, cnt_out.at[w])
    cbuf[pl.ds(0, 16)] = jnp.full((16,), off1, _i32)
    pltpu.sync_copy(cbuf, cnt_out.at[NW + w])


def _partition(src2d, dst2d):
    mesh = plsc.VectorSubcoreMesh(core_axis_name="c", subcore_axis_name="s")
    f = pl.kernel(
        _part_body,
        out_type=[
            jax.ShapeDtypeStruct((2 * ET,), _i32),
            jax.ShapeDtypeStruct((2 * ET,), _i32),
            jax.ShapeDtypeStruct((2 * NW, 16), _i32),
        ],
        mesh=mesh,
        scratch_types=[
            pltpu.VMEM((CROWS, 128), _i32),
            pltpu.VMEM((CROWS, 128), _i32),
            pltpu.VMEM((CAP + 16,), _i32),
            pltpu.VMEM((CAP + 16,), _i32),
            pltpu.VMEM((CAP + 16,), _i32),
            pltpu.VMEM((CAP + 16,), _i32),
            pltpu.VMEM((16,), _i32),
        ],
        compiler_params=pltpu.CompilerParams(needs_layout_passes=False,
                                             use_tc_tiling_on_sc=False),
    )
    return f(src2d, dst2d)


def _edge_body(h_hbm, as_hbm, ad_hbm, sp_hbm, dp_hbm, cnt_hbm,
               zacc_hbm, zden_hbm,
               acc_out, den_out,
               as_v, ad_v, sidx, didx, hrows, frows, exv, cnt_v,
               acc_sh, den_sh, gsem, ssem, isem):
    c = lax.axis_index("c")
    s = lax.axis_index("s")

    pltpu.sync_copy(zacc_hbm.at[pl.ds(s * STRIPE, STRIPE)],
                    acc_sh.at[pl.ds(s * STRIPE, STRIPE)])
    pltpu.sync_copy(zden_hbm.at[pl.ds(s * STRIPE, STRIPE)],
                    den_sh.at[pl.ds(s * STRIPE, STRIPE)])
    pltpu.sync_copy(as_hbm, as_v)
    pltpu.sync_copy(ad_hbm.at[pl.ds(c * HALF, HALF)], ad_v)
    pltpu.sync_copy(cnt_hbm, cnt_v)
    plsc.subcore_barrier()

    r0 = c * NW + 2 * s
    n0 = cnt_v[c * NW + 2 * s, pl.ds(0, 16)][0]
    n1 = cnt_v[c * NW + 2 * s + 1, pl.ds(0, 16)][0]
    nb0 = (n0 + EB - 1) // EB
    nb1 = (n1 + EB - 1) // EB
    total = nb0 + nb1

    def flatoff(b):
        return jnp.where(b < nb0,
                         r0 * CAP + b * EB,
                         (r0 + 1) * CAP + (b - nb0) * EB)

    def fire_idx(b, slot, sync):
        o = flatoff(b)
        if sync:
            pltpu.sync_copy(sp_hbm.at[pl.ds(o, EB)], sidx.at[slot])
            pltpu.sync_copy(dp_hbm.at[pl.ds(o, EB)], didx.at[slot])
        else:
            pltpu.async_copy(sp_hbm.at[pl.ds(o, EB)], sidx.at[slot], isem)
            pltpu.async_copy(dp_hbm.at[pl.ds(o, EB)], didx.at[slot], isem)

    def drain_idx():
        for _ in range(2):
            pltpu.make_async_copy(sp_hbm.at[pl.ds(0, EB)],
                                  sidx.at[0], isem).wait()

    def fire_gather(slot4, slot3):
        pltpu.async_copy(h_hbm.at[sidx.at[slot4]], hrows.at[slot3], gsem)

    def drain_gather():
        pltpu.make_async_copy(h_hbm.at[pl.ds(0, EB)],
                              hrows.at[0], gsem).wait()

    def fire_scatter(slot4, slot3):
        pltpu.async_copy(frows.at[slot3], acc_sh.at[didx.at[slot4]],
                         ssem, add=True)
        pltpu.async_copy(exv.at[pl.ds(slot3 * EB, EB)],
                         den_sh.at[didx.at[slot4]], ssem, add=True)

    def drain_scatter():
        pltpu.make_async_copy(zacc_hbm.at[pl.ds(0, EB)],
                              acc_sh.at[pl.ds(0, EB)], ssem).wait()
        pltpu.make_async_copy(zden_hbm.at[pl.ds(0, EB)],
                              den_sh.at[pl.ds(0, EB)], ssem).wait()

    @pl.when(total > 0)
    def _():
        fire_idx(0, 0, True)
        fire_gather(0, 0)

    @pl.when(total > 1)
    def _():
        fire_idx(1, 1, False)

    def block(b, carry):
        p3 = b % 3
        p4 = b % 4

        @pl.when(b >= 2)
        def _():
            drain_scatter()

        @pl.when(b + 2 < total)
        def _():
            fire_idx(b + 2, (b + 2) % 4, False)

        @pl.when(b + 1 < total)
        def _():
            drain_idx()
            fire_gather((b + 1) % 4, (b + 1) % 3)

        drain_gather()

        def grp(k, _):
            s16 = sidx[p4, pl.ds(k * 16, 16)]
            d16 = didx[p4, pl.ds(k * 16, 16)]
            a_s = plsc.load_gather(as_v, [s16])
            a_d = plsc.load_gather(ad_v, [d16])
            e = a_s + a_d
            e = jnp.where(e >= 0.0, e, 0.2 * e)
            exv[pl.ds(p3 * EB + k * 16, 16)] = jnp.exp(e)
            return 0
        lax.fori_loop(0, EB // 16, grp, 0, unroll=True)

        def scale(i, _):
            wv = plsc.load_gather(exv, [jnp.full((16,), p3 * EB + i, _i32)])
            for t in range(D // 32):
                pk = hrows[p3, i, pl.ds(t * 32, 32)]
                lo, hi = plsc.unpack(pk, format=plsc.PackFormat.INTERLEAVED)
                frows[p3, i, pl.ds(t * 32, 16)] = lo * wv
                frows[p3, i, pl.ds(t * 32 + 16, 16)] = hi * wv
            return 0
        lax.fori_loop(0, EB, scale, 0, unroll=4)

        fire_scatter(p4, p3)
        return carry

    lax.fori_loop(0, total, block, 0)

    @pl.when(total > 0)
    def _():
        drain_scatter()

    @pl.when(total > 1)
    def _():
        drain_scatter()

    plsc.subcore_barrier()

    pltpu.sync_copy(acc_sh.at[pl.ds(s * STRIPE, STRIPE)],
                    acc_out.at[c].at[pl.ds(s * STRIPE, STRIPE)])
    pltpu.sync_copy(den_sh.at[pl.ds(s * STRIPE, STRIPE)],
                    den_out.at[c].at[pl.ds(s * STRIPE, STRIPE)])


def _edge_pass(h, a_s, a_d, sp, dp, cnts, zacc, zden):
    mesh = plsc.VectorSubcoreMesh(core_axis_name="c", subcore_axis_name="s")
    f = pl.kernel(
        _edge_body,
        out_type=[
            jax.ShapeDtypeStruct((NC, HALF, D), _f32),
            jax.ShapeDtypeStruct((NC, HALF), _f32),
        ],
        mesh=mesh,
        scratch_types=[
            pltpu.VMEM((NACC,), _f32),
            pltpu.VMEM((HALF,), _f32),
            pltpu.VMEM((4, EB), _i32),
            pltpu.VMEM((4, EB), _i32),
            pltpu.VMEM((3, EB, D), jnp.bfloat16),
            pltpu.VMEM((3, EB, D), _f32),
            pltpu.VMEM((3 * EB,), _f32),
            pltpu.VMEM((2 * NW, 16), _i32),
            pltpu.VMEM_SHARED((HALF, D), _f32),
            pltpu.VMEM_SHARED((HALF,), _f32),
            pltpu.SemaphoreType.DMA,
            pltpu.SemaphoreType.DMA,
            pltpu.SemaphoreType.DMA,
        ],
        compiler_params=pltpu.CompilerParams(needs_layout_passes=False,
                                             use_tc_tiling_on_sc=False),
    )
    return f(h, a_s, a_d, sp, dp, cnts, zacc, zden)


def kernel(x, edge_index, batch, W1, a_src1, a_dst1, b1,
           W2, a_src2, a_dst2, b2, Wl1, bl1, Wl2, bl2):
    loops = jnp.arange(N, dtype=_i32)
    src = jnp.concatenate([edge_index[0].astype(_i32), loops])
    dst = jnp.concatenate([edge_index[1].astype(_i32), loops])
    src2d = jnp.pad(src, (0, ET - src.shape[0])).reshape(ER, 128)
    npad = ET - dst.shape[0]
    dpad = N + (jnp.arange(npad, dtype=_i32) % (NACC - N))
    dst2d = jnp.concatenate([dst, dpad]).reshape(ER, 128)
    xp = jnp.pad(x, ((0, NACC - N), (0, 0)))
    bb2d = jnp.pad(batch.astype(_i32), (0, NACC - N),
                   constant_values=NG).reshape(NACC // 128, 128)
    A1 = jnp.stack([a_src1, a_dst1], axis=1)
    A2 = jnp.stack([a_src2, a_dst2], axis=1)
    zacc = jnp.zeros((HALF, D), _f32)
    zden = jnp.zeros((HALF,), _f32)

    sp, dp, cnts = _partition(src2d, dst2d)
    h1, asad1 = _mm(xp, W1, A1)
    acc1, den1 = _edge_pass(h1, asad1[:, 0], asad1[:, 1], sp, dp, cnts,
                            zacc, zden)
    h2, asad2 = _mid(acc1.reshape(NACC, D), den1.reshape(NACC), b1, W2, A2)
    acc2, den2 = _edge_pass(h2, asad2[:, 0], asad2[:, 1], sp, dp, cnts,
                            zacc, zden)
    return _tail(acc2.reshape(NACC, D), den2.reshape(NACC), bb2d, b2,
                 Wl1, bl1, Wl2, bl2)


# speedup vs baseline: 1.0517x; 1.0517x over previous
import jax
import jax.numpy as jnp
from jax import lax
from jax.experimental import pallas as pl
from jax.experimental.pallas import tpu as pltpu
from jax.experimental.pallas import tpu_sc as plsc

N = 10000
E = 320000
D = 128
HID = 64
DOUT = 32
NG = 64

NC, NS = 2, 16
NW = NC * NS
HALF = 5120
EB = 96
ET = 344064
ER = ET // 128
CAP = ET // NW
CROWS = CAP // 128
NACC = 2 * HALF
STRIPE = HALF // NS
PADV = 2 * HALF - 1

_f32 = jnp.float32
_i32 = jnp.int32


def _selu(x):
    return 1.0507009873554805 * jnp.where(
        x > 0, x, 1.6732632423543772 * (jnp.exp(x) - 1.0))


def _interleave_bf16(h):
    t = h.reshape(h.shape[0], 4, 2, 16)
    return t.transpose(0, 1, 3, 2).reshape(h.shape[0], D).astype(
        jnp.bfloat16)


def _mm_body(x_ref, w_ref, a2_ref, h_ref, p_ref):
    i = pl.program_id(0)
    h = jnp.dot(x_ref[...], w_ref[...], preferred_element_type=_f32)
    h_ref[...] = _interleave_bf16(h)
    p = jnp.dot(h, a2_ref[...], preferred_element_type=_f32)
    row = i * x_ref.shape[0] + lax.broadcasted_iota(_i32, p.shape, 0)
    p_ref[...] = jnp.where(row < N, p, -1e30)


def _mm(x, W, A2):
    BR = 1024
    return pl.pallas_call(
        _mm_body,
        grid=(NACC // BR,),
        in_specs=[
            pl.BlockSpec((BR, D), lambda i: (i, 0)),
            pl.BlockSpec((D, D), lambda i: (0, 0)),
            pl.BlockSpec((D, 2), lambda i: (0, 0)),
        ],
        out_specs=[
            pl.BlockSpec((BR, D), lambda i: (i, 0)),
            pl.BlockSpec((BR, 2), lambda i: (i, 0)),
        ],
        out_shape=[
            jax.ShapeDtypeStruct((NACC, D), jnp.bfloat16),
            jax.ShapeDtypeStruct((NACC, 2), _f32),
        ],
    )(x, W, A2)


def _mid_body(acc_ref, d_ref, b_ref, w_ref, a2_ref, h_ref, p_ref):
    i = pl.program_id(0)
    den = d_ref[...] + 1e-16
    z = _selu(acc_ref[...] / den + b_ref[...])
    row = i * acc_ref.shape[0] + lax.broadcasted_iota(_i32, z.shape, 0)
    z = jnp.where(row < N, z, 0.0)
    h = jnp.dot(z, w_ref[...], preferred_element_type=_f32)
    h_ref[...] = _interleave_bf16(h)
    p = jnp.dot(h, a2_ref[...], preferred_element_type=_f32)
    prow = i * z.shape[0] + lax.broadcasted_iota(_i32, p.shape, 0)
    p_ref[...] = jnp.where(prow < N, p, -1e30)


def _mid(acc, den, b, W, A2):
    BR = 1024
    return pl.pallas_call(
        _mid_body,
        grid=(NACC // BR,),
        in_specs=[
            pl.BlockSpec((BR, D), lambda i: (i, 0)),
            pl.BlockSpec((BR, 1), lambda i: (i, 0)),
            pl.BlockSpec((1, D), lambda i: (0, 0)),
            pl.BlockSpec((D, D), lambda i: (0, 0)),
            pl.BlockSpec((D, 2), lambda i: (0, 0)),
        ],
        out_specs=[
            pl.BlockSpec((BR, D), lambda i: (i, 0)),
            pl.BlockSpec((BR, 2), lambda i: (i, 0)),
        ],
        out_shape=[
            jax.ShapeDtypeStruct((NACC, D), jnp.bfloat16),
            jax.ShapeDtypeStruct((NACC, 2), _f32),
        ],
    )(acc, den.reshape(NACC, 1), b.reshape(1, D), W, A2)


def _tail_body(acc_ref, d_ref, bb_ref, b_ref,
               wl1_ref, bl1_ref, wl2_ref, bl2_ref, out_ref,
               sum_ref, cnt_ref):
    i = pl.program_id(0)
    BR = acc_ref.shape[0]
    den = d_ref[...] + 1e-16
    z = _selu(acc_ref[...] / den + b_ref[...])
    row = i * BR + lax.broadcasted_iota(_i32, z.shape, 0)
    z = jnp.where(row < N, z, 0.0)

    bb = bb_ref[...].reshape(1, BR)
    gid = lax.broadcasted_iota(_i32, (NG, BR), 0)
    msk = (bb == gid).astype(_f32)

    @pl.when(i == 0)
    def _():
        sum_ref[...] = jnp.zeros_like(sum_ref)
        cnt_ref[...] = jnp.zeros_like(cnt_ref)

    sum_ref[...] += jnp.dot(msk, z, preferred_element_type=_f32)
    cnt_ref[...] += jnp.sum(msk, axis=1, keepdims=True)

    @pl.when(i == pl.num_programs(0) - 1)
    def _():
        g = sum_ref[...] / jnp.maximum(cnt_ref[...], 1.0)
        g = _selu(g)
        g = _selu(jnp.dot(g, wl1_ref[...], preferred_element_type=_f32)
                  + bl1_ref[...])
        lg = jnp.dot(g, wl2_ref[...], preferred_element_type=_f32) \
            + bl2_ref[...]
        m = jnp.max(lg, axis=1, keepdims=True)
        lse = jnp.log(jnp.sum(jnp.exp(lg - m), axis=1, keepdims=True))
        out_ref[...] = lg - m - lse


def _tail(acc, den, bb2d, b, Wl1, bl1, Wl2, bl2):
    BR = 1024
    return pl.pallas_call(
        _tail_body,
        grid=(NACC // BR,),
        in_specs=[
            pl.BlockSpec((BR, D), lambda i: (i, 0)),
            pl.BlockSpec((BR, 1), lambda i: (i, 0)),
            pl.BlockSpec((8, 128), lambda i: (i, 0)),
            pl.BlockSpec((1, D), lambda i: (0, 0)),
            pl.BlockSpec((D, HID), lambda i: (0, 0)),
            pl.BlockSpec((1, HID), lambda i: (0, 0)),
            pl.BlockSpec((HID, DOUT), lambda i: (0, 0)),
            pl.BlockSpec((1, DOUT), lambda i: (0, 0)),
        ],
        out_specs=pl.BlockSpec((NG, DOUT), lambda i: (0, 0)),
        out_shape=jax.ShapeDtypeStruct((NG, DOUT), _f32),
        scratch_shapes=[
            pltpu.VMEM((NG, D), _f32),
            pltpu.VMEM((NG, 1), _f32),
        ],
    )(acc, den.reshape(NACC, 1), bb2d, b.reshape(1, D),
      Wl1, bl1.reshape(1, HID), Wl2, bl2.reshape(1, DOUT))


def _part_body(s_hbm, d_hbm, sp_out, dp_out, cnt_out,
               s_in, d_in, ls0, ld0, ls1, ld1, cbuf):
    c = lax.axis_index("c")
    s = lax.axis_index("s")
    w = c * NS + s

    pltpu.sync_copy(s_hbm.at[pl.ds(w * CROWS, CROWS)], s_in)
    pltpu.sync_copy(d_hbm.at[pl.ds(w * CROWS, CROWS)], d_in)

    padsrc = jnp.full((16,), PADV, _i32)

    def init(k, _):
        spread = (k % (HALF // 16)) * 16 + jnp.arange(16, dtype=_i32)
        ls0[pl.ds(k * 16, 16)] = padsrc
        ls1[pl.ds(k * 16, 16)] = padsrc
        ld0[pl.ds(k * 16, 16)] = spread
        ld1[pl.ds(k * 16, 16)] = spread
        return 0
    lax.fori_loop(0, CAP // 16, init, 0)

    def compact(k, carry):
        off0, off1 = carry
        s16 = s_in[k // 8, pl.ds((k % 8) * 16, 16)]
        d16 = d_in[k // 8, pl.ds((k % 8) * 16, 16)]
        m0 = d16 < HALF
        m1 = jnp.logical_not(m0)
        plsc.store_compressed(ls0.at[pl.ds(off0, 16)], s16, mask=m0)
        plsc.store_compressed(ld0.at[pl.ds(off0, 16)], d16, mask=m0)
        plsc.store_compressed(ls1.at[pl.ds(off1, 16)], s16, mask=m1)
        plsc.store_compressed(ld1.at[pl.ds(off1, 16)], d16 - HALF, mask=m1)
        n0 = plsc.all_reduce_population_count(m0)[0]
        return (off0 + n0, off1 + (16 - n0))
    off0, off1 = lax.fori_loop(0, CAP // 16, compact, (0, 0))

    pltpu.sync_copy(ls0.at[pl.ds(0, CAP)], sp_out.at[pl.ds(w * CAP, CAP)])
    pltpu.sync_copy(ld0.at[pl.ds(0, CAP)], dp_out.at[pl.ds(w * CAP, CAP)])
    pltpu.sync_copy(ls1.at[pl.ds(0, CAP)],
                    sp_out.at[pl.ds((NW + w) * CAP, CAP)])
    pltpu.sync_copy(ld1.at[pl.ds(0, CAP)],
                    dp_out.at[pl.ds((NW + w) * CAP, CAP)])

    cbuf[pl.ds(0, 16)] = jnp.full((16,), off0, _i32)
    pltpu.sync_copy(cbuf, cnt_out.at[w])
    cbuf[pl.ds(0, 16)] = jnp.full((16,), off1, _i32)
    pltpu.sync_copy(cbuf, cnt_out.at[NW + w])


def _partition(src2d, dst2d):
    mesh = plsc.VectorSubcoreMesh(core_axis_name="c", subcore_axis_name="s")
    f = pl.kernel(
        _part_body,
        out_type=[
            jax.ShapeDtypeStruct((2 * ET,), _i32),
            jax.ShapeDtypeStruct((2 * ET,), _i32),
            jax.ShapeDtypeStruct((2 * NW, 16), _i32),
        ],
        mesh=mesh,
        scratch_types=[
            pltpu.VMEM((CROWS, 128), _i32),
            pltpu.VMEM((CROWS, 128), _i32),
            pltpu.VMEM((CAP + 16,), _i32),
            pltpu.VMEM((CAP + 16,), _i32),
            pltpu.VMEM((CAP + 16,), _i32),
            pltpu.VMEM((CAP + 16,), _i32),
            pltpu.VMEM((16,), _i32),
        ],
        compiler_params=pltpu.CompilerParams(needs_layout_passes=False,
                                             use_tc_tiling_on_sc=False),
    )
    return f(src2d, dst2d)


def _edge_body(h_hbm, as_hbm, ad_hbm, sp_hbm, dp_hbm, cnt_hbm,
               zacc_hbm, zden_hbm,
               acc_out, den_out,
               as_v, ad_v, sidx, didx, hrows, frows, exv, cnt_v,
               acc_sh, den_sh, gsem, ssem, isem):
    c = lax.axis_index("c")
    s = lax.axis_index("s")

    pltpu.sync_copy(zacc_hbm.at[pl.ds(s * STRIPE, STRIPE)],
                    acc_sh.at[pl.ds(s * STRIPE, STRIPE)])
    pltpu.sync_copy(zden_hbm.at[pl.ds(s * STRIPE, STRIPE)],
                    den_sh.at[pl.ds(s * STRIPE, STRIPE)])
    pltpu.sync_copy(as_hbm, as_v)
    pltpu.sync_copy(ad_hbm.at[pl.ds(c * HALF, HALF)], ad_v)
    pltpu.sync_copy(cnt_hbm, cnt_v)
    plsc.subcore_barrier()

    r0 = c * NW + 2 * s
    n0 = cnt_v[c * NW + 2 * s, pl.ds(0, 16)][0]
    n1 = cnt_v[c * NW + 2 * s + 1, pl.ds(0, 16)][0]
    nb0 = (n0 + EB - 1) // EB
    nb1 = (n1 + EB - 1) // EB
    total = nb0 + nb1

    def flatoff(b):
        return jnp.where(b < nb0,
                         r0 * CAP + b * EB,
                         (r0 + 1) * CAP + (b - nb0) * EB)

    def fire_idx(b, slot, sync):
        o = flatoff(b)
        if sync:
            pltpu.sync_copy(sp_hbm.at[pl.ds(o, EB)], sidx.at[slot])
            pltpu.sync_copy(dp_hbm.at[pl.ds(o, EB)], didx.at[slot])
        else:
            pltpu.async_copy(sp_hbm.at[pl.ds(o, EB)], sidx.at[slot], isem)
            pltpu.async_copy(dp_hbm.at[pl.ds(o, EB)], didx.at[slot], isem)

    def drain_idx():
        for _ in range(2):
            pltpu.make_async_copy(sp_hbm.at[pl.ds(0, EB)],
                                  sidx.at[0], isem).wait()

    def fire_gather(slot4, slot3):
        pltpu.async_copy(h_hbm.at[sidx.at[slot4]], hrows.at[slot3], gsem)

    def drain_gather():
        pltpu.make_async_copy(h_hbm.at[pl.ds(0, EB)],
                              hrows.at[0], gsem).wait()

    def fire_scatter(slot4, slot3):
        pltpu.async_copy(frows.at[slot3], acc_sh.at[didx.at[slot4]],
                         ssem, add=True)
        pltpu.async_copy(exv.at[pl.ds(slot3 * EB, EB)],
                         den_sh.at[didx.at[slot4]], ssem, add=True)

    def drain_scatter():
        pltpu.make_async_copy(zacc_hbm.at[pl.ds(0, EB)],
                              acc_sh.at[pl.ds(0, EB)], ssem).wait()
        pltpu.make_async_copy(zden_hbm.at[pl.ds(0, EB)],
                              den_sh.at[pl.ds(0, EB)], ssem).wait()

    @pl.when(total > 0)
    def _():
        fire_idx(0, 0, True)
        fire_gather(0, 0)

    @pl.when(total > 1)
    def _():
        fire_idx(1, 1, False)

    def block(b, carry):
        p3 = b % 2
        p4 = b % 4

        @pl.when(b >= 2)
        def _():
            drain_scatter()

        @pl.when(b + 2 < total)
        def _():
            fire_idx(b + 2, (b + 2) % 4, False)

        @pl.when(b + 1 < total)
        def _():
            drain_idx()
            fire_gather((b + 1) % 4, (b + 1) % 2)

        drain_gather()

        def grp(k, _):
            s16 = sidx[p4, pl.ds(k * 16, 16)]
            d16 = didx[p4, pl.ds(k * 16, 16)]
            a_s = plsc.load_gather(as_v, [s16])
            a_d = plsc.load_gather(ad_v, [d16])
            e = a_s + a_d
            e = jnp.where(e >= 0.0, e, 0.2 * e)
            exv[pl.ds(p3 * EB + k * 16, 16)] = jnp.exp(e)
            return 0
        lax.fori_loop(0, EB // 16, grp, 0, unroll=True)

        def scale(i, _):
            wv = plsc.load_gather(exv, [jnp.full((16,), p3 * EB + i, _i32)])
            for t in range(D // 32):
                pk = hrows[p3, i, pl.ds(t * 32, 32)]
                lo, hi = plsc.unpack(pk, format=plsc.PackFormat.INTERLEAVED)
                frows[p3, i, pl.ds(t * 32, 16)] = lo * wv
                frows[p3, i, pl.ds(t * 32 + 16, 16)] = hi * wv
            return 0
        lax.fori_loop(0, EB, scale, 0, unroll=4)

        fire_scatter(p4, p3)
        return carry

    lax.fori_loop(0, total, block, 0)

    @pl.when(total > 0)
    def _():
        drain_scatter()

    @pl.when(total > 1)
    def _():
        drain_scatter()

    plsc.subcore_barrier()

    pltpu.sync_copy(acc_sh.at[pl.ds(s * STRIPE, STRIPE)],
                    acc_out.at[c].at[pl.ds(s * STRIPE, STRIPE)])
    pltpu.sync_copy(den_sh.at[pl.ds(s * STRIPE, STRIPE)],
                    den_out.at[c].at[pl.ds(s * STRIPE, STRIPE)])


def _edge_pass(h, a_s, a_d, sp, dp, cnts, zacc, zden):
    mesh = plsc.VectorSubcoreMesh(core_axis_name="c", subcore_axis_name="s")
    f = pl.kernel(
        _edge_body,
        out_type=[
            jax.ShapeDtypeStruct((NC, HALF, D), _f32),
            jax.ShapeDtypeStruct((NC, HALF), _f32),
        ],
        mesh=mesh,
        scratch_types=[
            pltpu.VMEM((NACC,), _f32),
            pltpu.VMEM((HALF,), _f32),
            pltpu.VMEM((4, EB), _i32),
            pltpu.VMEM((4, EB), _i32),
            pltpu.VMEM((2, EB, D), jnp.bfloat16),
            pltpu.VMEM((2, EB, D), _f32),
            pltpu.VMEM((2 * EB,), _f32),
            pltpu.VMEM((2 * NW, 16), _i32),
            pltpu.VMEM_SHARED((HALF, D), _f32),
            pltpu.VMEM_SHARED((HALF,), _f32),
            pltpu.SemaphoreType.DMA,
            pltpu.SemaphoreType.DMA,
            pltpu.SemaphoreType.DMA,
        ],
        compiler_params=pltpu.CompilerParams(needs_layout_passes=False,
                                             use_tc_tiling_on_sc=False),
    )
    return f(h, a_s, a_d, sp, dp, cnts, zacc, zden)


def kernel(x, edge_index, batch, W1, a_src1, a_dst1, b1,
           W2, a_src2, a_dst2, b2, Wl1, bl1, Wl2, bl2):
    loops = jnp.arange(N, dtype=_i32)
    src = jnp.concatenate([edge_index[0].astype(_i32), loops])
    dst = jnp.concatenate([edge_index[1].astype(_i32), loops])
    src2d = jnp.pad(src, (0, ET - src.shape[0])).reshape(ER, 128)
    npad = ET - dst.shape[0]
    dpad = N + (jnp.arange(npad, dtype=_i32) % (NACC - N))
    dst2d = jnp.concatenate([dst, dpad]).reshape(ER, 128)
    xp = jnp.pad(x, ((0, NACC - N), (0, 0)))
    bb2d = jnp.pad(batch.astype(_i32), (0, NACC - N),
                   constant_values=NG).reshape(NACC // 128, 128)
    A1 = jnp.stack([a_src1, a_dst1], axis=1)
    A2 = jnp.stack([a_src2, a_dst2], axis=1)
    zacc = jnp.zeros((HALF, D), _f32)
    zden = jnp.zeros((HALF,), _f32)

    sp, dp, cnts = _partition(src2d, dst2d)
    h1, asad1 = _mm(xp, W1, A1)
    acc1, den1 = _edge_pass(h1, asad1[:, 0], asad1[:, 1], sp, dp, cnts,
                            zacc, zden)
    h2, asad2 = _mid(acc1.reshape(NACC, D), den1.reshape(NACC), b1, W2, A2)
    acc2, den2 = _edge_pass(h2, asad2[:, 0], asad2[:, 1], sp, dp, cnts,
                            zacc, zden)
    return _tail(acc2.reshape(NACC, D), den2.reshape(NACC), bb2d, b2,
                 Wl1, bl1, Wl2, bl2)


# speedup vs baseline: 1.4262x; 1.3561x over previous
import jax
import jax.numpy as jnp
from jax import lax
from jax.experimental import pallas as pl
from jax.experimental.pallas import tpu as pltpu
from jax.experimental.pallas import tpu_sc as plsc

N = 10000
E = 320000
D = 128
DH = D // 2
HID = 64
DOUT = 32
NG = 64

NC, NS = 2, 16
EB = 256
EBJ = EB // 128
NBLK = 84
EW = EB * NBLK
ET = EW * NS
ER = ET // 128
NACC = 10240
STRIPE = NACC // NS

_f32 = jnp.float32


def _selu(x):
    return 1.0507009873554805 * jnp.where(
        x > 0, x, 1.6732632423543772 * (jnp.exp(x) - 1.0))


def _mm_body(x_ref, w_ref, a2_ref, h_ref, p_ref):
    h = jnp.dot(x_ref[...], w_ref[...], preferred_element_type=_f32)
    h_ref[0] = h[:, :DH]
    h_ref[1] = h[:, DH:]
    p_ref[...] = jnp.dot(h, a2_ref[...], preferred_element_type=_f32)


def _mm(x, W, A2):
    BR = 1024
    return pl.pallas_call(
        _mm_body,
        grid=(NACC // BR,),
        in_specs=[
            pl.BlockSpec((BR, D), lambda i: (i, 0)),
            pl.BlockSpec((D, D), lambda i: (0, 0)),
            pl.BlockSpec((D, 2), lambda i: (0, 0)),
        ],
        out_specs=[
            pl.BlockSpec((NC, BR, DH), lambda i: (0, i, 0)),
            pl.BlockSpec((BR, 2), lambda i: (i, 0)),
        ],
        out_shape=[
            jax.ShapeDtypeStruct((NC, NACC, DH), _f32),
            jax.ShapeDtypeStruct((NACC, 2), _f32),
        ],
    )(x, W, A2)


def _mid_body(a0_ref, a1_ref, d_ref, b_ref, w_ref, a2_ref, h_ref, p_ref):
    i = pl.program_id(0)
    den = d_ref[...] + 1e-16
    acc = jnp.concatenate([a0_ref[...], a1_ref[...]], axis=1)
    z = _selu(acc / den + b_ref[...])
    row = i * a0_ref.shape[0] + lax.broadcasted_iota(jnp.int32, z.shape, 0)
    z = jnp.where(row < N, z, 0.0)
    h = jnp.dot(z, w_ref[...], preferred_element_type=_f32)
    h_ref[0] = h[:, :DH]
    h_ref[1] = h[:, DH:]
    p_ref[...] = jnp.dot(h, a2_ref[...], preferred_element_type=_f32)


def _mid(acc, den, b, W, A2):
    BR = 1024
    return pl.pallas_call(
        _mid_body,
        grid=(NACC // BR,),
        in_specs=[
            pl.BlockSpec((BR, DH), lambda i: (i, 0)),
            pl.BlockSpec((BR, DH), lambda i: (i, 0)),
            pl.BlockSpec((BR, 1), lambda i: (i, 0)),
            pl.BlockSpec((1, D), lambda i: (0, 0)),
            pl.BlockSpec((D, D), lambda i: (0, 0)),
            pl.BlockSpec((D, 2), lambda i: (0, 0)),
        ],
        out_specs=[
            pl.BlockSpec((NC, BR, DH), lambda i: (0, i, 0)),
            pl.BlockSpec((BR, 2), lambda i: (i, 0)),
        ],
        out_shape=[
            jax.ShapeDtypeStruct((NC, NACC, DH), _f32),
            jax.ShapeDtypeStruct((NACC, 2), _f32),
        ],
    )(acc[0], acc[1], den.reshape(NACC, 1), b.reshape(1, D), W, A2)


def _tail_body(a0_ref, a1_ref, d_ref, bb_ref, b_ref,
               wl1_ref, bl1_ref, wl2_ref, bl2_ref, out_ref,
               sum_ref, cnt_ref):
    i = pl.program_id(0)
    BR = a0_ref.shape[0]
    den = d_ref[...] + 1e-16
    acc = jnp.concatenate([a0_ref[...], a1_ref[...]], axis=1)
    z = _selu(acc / den + b_ref[...])
    row = i * BR + lax.broadcasted_iota(jnp.int32, z.shape, 0)
    z = jnp.where(row < N, z, 0.0)

    bb = bb_ref[...].reshape(1, BR)
    gid = lax.broadcasted_iota(jnp.int32, (NG, BR), 0)
    msk = (bb == gid).astype(_f32)

    @pl.when(i == 0)
    def _():
        sum_ref[...] = jnp.zeros_like(sum_ref)
        cnt_ref[...] = jnp.zeros_like(cnt_ref)

    sum_ref[...] += jnp.dot(msk, z, preferred_element_type=_f32)
    cnt_ref[...] += jnp.sum(msk, axis=1, keepdims=True)

    @pl.when(i == pl.num_programs(0) - 1)
    def _():
        g = sum_ref[...] / jnp.maximum(cnt_ref[...], 1.0)
        g = _selu(g)
        g = _selu(jnp.dot(g, wl1_ref[...], preferred_element_type=_f32)
                  + bl1_ref[...])
        lg = jnp.dot(g, wl2_ref[...], preferred_element_type=_f32) \
            + bl2_ref[...]
        m = jnp.max(lg, axis=1, keepdims=True)
        lse = jnp.log(jnp.sum(jnp.exp(lg - m), axis=1, keepdims=True))
        out_ref[...] = lg - m - lse


def _tail(acc, den, bb2d, b, Wl1, bl1, Wl2, bl2):
    BR = 1024
    return pl.pallas_call(
        _tail_body,
        grid=(NACC // BR,),
        in_specs=[
            pl.BlockSpec((BR, DH), lambda i: (i, 0)),
            pl.BlockSpec((BR, DH), lambda i: (i, 0)),
            pl.BlockSpec((BR, 1), lambda i: (i, 0)),
            pl.BlockSpec((8, 128), lambda i: (i, 0)),
            pl.BlockSpec((1, D), lambda i: (0, 0)),
            pl.BlockSpec((D, HID), lambda i: (0, 0)),
            pl.BlockSpec((1, HID), lambda i: (0, 0)),
            pl.BlockSpec((HID, DOUT), lambda i: (0, 0)),
            pl.BlockSpec((1, DOUT), lambda i: (0, 0)),
        ],
        out_specs=pl.BlockSpec((NG, DOUT), lambda i: (0, 0)),
        out_shape=jax.ShapeDtypeStruct((NG, DOUT), _f32),
        scratch_shapes=[
            pltpu.VMEM((NG, D), _f32),
            pltpu.VMEM((NG, 1), _f32),
        ],
    )(acc[0], acc[1], den.reshape(NACC, 1), bb2d, b.reshape(1, D),
      Wl1, bl1.reshape(1, HID), Wl2, bl2.reshape(1, DOUT))


def _edge_body(h_hbm, as_hbm, ad_hbm, src_hbm, dst_hbm, zacc_hbm, zden_hbm,
               acc_out, den_out,
               as_v, ad_v, sidx, didx, hrows, exv, acc_sh, den_sh,
               gsem, ssem, isem):
    c = lax.axis_index("c")
    s = lax.axis_index("s")

    pltpu.sync_copy(zacc_hbm.at[pl.ds(s * STRIPE, STRIPE)],
                    acc_sh.at[pl.ds(s * STRIPE, STRIPE)])
    pltpu.sync_copy(zden_hbm.at[pl.ds(s * STRIPE, STRIPE)],
                    den_sh.at[pl.ds(s * STRIPE, STRIPE)])
    pltpu.sync_copy(as_hbm, as_v)
    pltpu.sync_copy(ad_hbm, ad_v)
    plsc.subcore_barrier()

    row0 = s * (EW // 128)

    def fire_idx(b, slot, sync):
        r = row0 + b * EBJ
        if sync:
            pltpu.sync_copy(src_hbm.at[pl.ds(r, EBJ)], sidx.at[slot])
            pltpu.sync_copy(dst_hbm.at[pl.ds(r, EBJ)], didx.at[slot])
        else:
            pltpu.async_copy(src_hbm.at[pl.ds(r, EBJ)], sidx.at[slot], isem)
            pltpu.async_copy(dst_hbm.at[pl.ds(r, EBJ)], didx.at[slot], isem)

    def drain_idx():
        for _ in range(2):
            pltpu.make_async_copy(src_hbm.at[pl.ds(0, EBJ)],
                                  sidx.at[0], isem).wait()

    def fire_gather(slot4, slot3):
        for j in range(EBJ):
            pltpu.async_copy(h_hbm.at[c].at[sidx.at[slot4].at[j]],
                             hrows.at[slot3].at[pl.ds(j * 128, 128)], gsem)

    def drain_gather():
        pltpu.make_async_copy(zacc_hbm.at[pl.ds(0, EB)],
                              hrows.at[0], gsem).wait()

    def fire_scatter(slot4, slot3):
        for j in range(EBJ):
            pltpu.async_copy(hrows.at[slot3].at[pl.ds(j * 128, 128)],
                             acc_sh.at[didx.at[slot4].at[j]], ssem, add=True)

        @pl.when(c == 0)
        def _():
            for j in range(EBJ):
                pltpu.async_copy(exv.at[pl.ds(slot3 * EB + j * 128, 128)],
                                 den_sh.at[didx.at[slot4].at[j]], ssem,
                                 add=True)

    def drain_scatter():
        pltpu.make_async_copy(zacc_hbm.at[pl.ds(0, EB)],
                              acc_sh.at[pl.ds(0, EB)], ssem).wait()

        @pl.when(c == 0)
        def _():
            pltpu.make_async_copy(zden_hbm.at[pl.ds(0, EB)],
                                  den_sh.at[pl.ds(0, EB)], ssem).wait()

    fire_idx(0, 0, True)
    fire_idx(1, 1, False)
    fire_gather(0, 0)

    def block(b, carry):
        p3 = b % 3
        p4 = b % 4

        @pl.when(b >= 2)
        def _():
            drain_scatter()

        @pl.when(b + 2 < NBLK)
        def _():
            fire_idx(b + 2, (b + 2) % 4, False)

        @pl.when(b + 1 < NBLK)
        def _():
            drain_idx()
            fire_gather((b + 1) % 4, (b + 1) % 3)

        drain_gather()

        for j in range(EBJ):
            def grp(k, _):
                s16 = sidx[p4, j, pl.ds(k * 16, 16)]
                d16 = didx[p4, j, pl.ds(k * 16, 16)]
                a_s = plsc.load_gather(as_v, [s16])
                a_d = plsc.load_gather(ad_v, [d16])
                e = a_s + a_d
                e = jnp.where(e >= 0.0, e, 0.2 * e)
                exv[pl.ds(p3 * EB + j * 128 + k * 16, 16)] = jnp.exp(e)
                return 0
            lax.fori_loop(0, 8, grp, 0, unroll=True)

        def scale(i, _):
            w = plsc.load_gather(exv, [jnp.full((16,), p3 * EB + i,
                                                jnp.int32)])
            for t in range(DH // 16):
                hrows[p3, i, pl.ds(t * 16, 16)] = \
                    hrows[p3, i, pl.ds(t * 16, 16)] * w
            return 0
        lax.fori_loop(0, EB, scale, 0, unroll=4)

        fire_scatter(p4, p3)
        return carry

    lax.fori_loop(0, NBLK, block, 0)
    drain_scatter()
    drain_scatter()
    plsc.subcore_barrier()

    pltpu.sync_copy(acc_sh.at[pl.ds(s * STRIPE, STRIPE)],
                    acc_out.at[c].at[pl.ds(s * STRIPE, STRIPE)])

    @pl.when(c == 0)
    def _():
        pltpu.sync_copy(den_sh.at[pl.ds(s * STRIPE, STRIPE)],
                        den_out.at[pl.ds(s * STRIPE, STRIPE)])


def _edge_pass(h3, a_s, a_d, src2d, dst2d, zacc, zden):
    mesh = plsc.VectorSubcoreMesh(core_axis_name="c", subcore_axis_name="s")
    f = pl.kernel(
        _edge_body,
        out_type=[
            jax.ShapeDtypeStruct((NC, NACC, DH), _f32),
            jax.ShapeDtypeStruct((NACC,), _f32),
        ],
        mesh=mesh,
        scratch_types=[
            pltpu.VMEM((NACC,), _f32),
            pltpu.VMEM((NACC,), _f32),
            pltpu.VMEM((4, EBJ, 128), jnp.int32),
            pltpu.VMEM((4, EBJ, 128), jnp.int32),
            pltpu.VMEM((3, EB, DH), _f32),
            pltpu.VMEM((3 * EB,), _f32),
            pltpu.VMEM_SHARED((NACC, DH), _f32),
            pltpu.VMEM_SHARED((NACC,), _f32),
            pltpu.SemaphoreType.DMA,
            pltpu.SemaphoreType.DMA,
            pltpu.SemaphoreType.DMA,
        ],
        compiler_params=pltpu.CompilerParams(needs_layout_passes=False,
                                             use_tc_tiling_on_sc=False),
    )
    return f(h3, a_s, a_d, src2d, dst2d, zacc, zden)


def kernel(x, edge_index, batch, W1, a_src1, a_dst1, b1,
           W2, a_src2, a_dst2, b2, Wl1, bl1, Wl2, bl2):
    loops = jnp.arange(N, dtype=jnp.int32)
    src = jnp.concatenate([edge_index[0].astype(jnp.int32), loops])
    dst = jnp.concatenate([edge_index[1].astype(jnp.int32), loops])
    src2d = jnp.pad(src, (0, ET - src.shape[0])).reshape(ER, 128)
    dst2d = jnp.pad(dst, (0, ET - dst.shape[0]),
                    constant_values=N).reshape(ER, 128)
    xp = jnp.pad(x, ((0, NACC - N), (0, 0)))
    bb2d = jnp.pad(batch.astype(jnp.int32), (0, NACC - N),
                   constant_values=NG).reshape(NACC // 128, 128)
    A1 = jnp.stack([a_src1, a_dst1], axis=1)
    A2 = jnp.stack([a_src2, a_dst2], axis=1)
    zacc = jnp.zeros((NACC, DH), _f32)
    zden = jnp.zeros((NACC,), _f32)

    h1, asad1 = _mm(xp, W1, A1)
    acc1, den1 = _edge_pass(h1, asad1[:, 0], asad1[:, 1], src2d, dst2d,
                            zacc, zden)
    h2, asad2 = _mid(acc1, den1, b1, W2, A2)
    acc2, den2 = _edge_pass(h2, asad2[:, 0], asad2[:, 1], src2d, dst2d,
                            zacc, zden)
    return _tail(acc2, den2, bb2d, b2, Wl1, bl1, Wl2, bl2)
